# baseline (speedup 1.0000x reference)
import jax
import jax.numpy as jnp
from jax.experimental import pallas as pl
from jax.experimental.pallas import tpu as pltpu
from jax.experimental.pallas import tpu_sc as plsc

_R = 8
_C = 128
_LANES = 16


def kernel(x, pos_embed_weight):
    batch, seq, d = x.shape
    rows = batch * seq
    xf = x.reshape(rows, d)
    w = pos_embed_weight[:seq]
    n_r = rows // _R
    n_rw = seq // _R
    n_c = d // _C

    mesh = plsc.VectorSubcoreMesh(core_axis_name="core", subcore_axis_name="subcore")

    @pl.kernel(out_type=jax.ShapeDtypeStruct((rows, d), x.dtype), mesh=mesh)
    def sc_add(x_hbm, w_hbm, o_hbm):
        def body(x_vmem, w_vmem, o_vmem):
            @pl.loop(0, _R)
            def _(r):
                @pl.loop(0, _C, step=_LANES)
                def _(c):
                    slc = (pl.ds(r, 1), pl.ds(c, _LANES))
                    o_vmem.at[*slc][...] = x_vmem.at[*slc][...] + w_vmem.at[*slc][...]

        pltpu.emit_pipeline(
            body,
            grid=(n_r, n_c),
            in_specs=[
                pl.BlockSpec((_R, _C), lambda i, j: (i, j)),
                pl.BlockSpec((_R, _C), lambda i, j: (jax.lax.rem(i, n_rw), j)),
            ],
            out_specs=[pl.BlockSpec((_R, _C), lambda i, j: (i, j))],
            core_axis_name=("core", "subcore"),
            dimension_semantics=(pltpu.PARALLEL, pltpu.PARALLEL),
        )(x_hbm, w_hbm, o_hbm)

    return sc_add(xf, w).reshape(batch, seq, d)


# speedup vs baseline: 1.3776x; 1.3776x over previous
import jax
import jax.numpy as jnp
from jax.experimental import pallas as pl
from jax.experimental.pallas import tpu as pltpu
from jax.experimental.pallas import tpu_sc as plsc

_R = 16
_LANES = 16


def kernel(x, pos_embed_weight):
    batch, seq, d = x.shape
    rows = batch * seq
    xf = x.reshape(rows, d)
    w = pos_embed_weight[:seq]
    n_r = rows // _R
    n_rw = seq // _R

    mesh = plsc.VectorSubcoreMesh(core_axis_name="core", subcore_axis_name="subcore")

    @pl.kernel(out_type=jax.ShapeDtypeStruct((rows, d), x.dtype), mesh=mesh)
    def sc_add(x_hbm, w_hbm, o_hbm):
        def body(x_vmem, w_vmem, o_vmem):
            @pl.loop(0, _R)
            def _(r):
                @pl.loop(0, d, step=_LANES, unroll=8)
                def _(c):
                    slc = (pl.ds(r, 1), pl.ds(c, _LANES))
                    o_vmem.at[*slc][...] = x_vmem.at[*slc][...] + w_vmem.at[*slc][...]

        pltpu.emit_pipeline(
            body,
            grid=(n_r,),
            in_specs=[
                pl.BlockSpec((_R, d), lambda i: (i, 0)),
                pl.BlockSpec((_R, d), lambda i: (jax.lax.rem(i, n_rw), 0)),
            ],
            out_specs=[pl.BlockSpec((_R, d), lambda i: (i, 0))],
            core_axis_name=("core", "subcore"),
            dimension_semantics=(pltpu.PARALLEL,),
        )(x_hbm, w_hbm, o_hbm)

    return sc_add(xf, w).reshape(batch, seq, d)


# speedup vs baseline: 5.4853x; 3.9819x over previous
import jax
import jax.numpy as jnp
from jax.experimental import pallas as pl
from jax.experimental.pallas import tpu as pltpu


def _add_body(x_ref, w_ref, o_ref):
    o_ref[...] = x_ref[...] + w_ref[...][None, :, :]


def kernel(x, pos_embed_weight):
    batch, seq, d = x.shape
    s_blk = min(2048, seq)
    n_seq = seq // s_blk
    grid = (n_seq, batch)
    out = pl.pallas_call(
        _add_body,
        grid=grid,
        in_specs=[
            pl.BlockSpec((1, s_blk, d), lambda i, j: (j, i, 0)),
            pl.BlockSpec((s_blk, d), lambda i, j: (i, 0)),
        ],
        out_specs=pl.BlockSpec((1, s_blk, d), lambda i, j: (j, i, 0)),
        out_shape=jax.ShapeDtypeStruct((batch, seq, d), x.dtype),
        compiler_params=pltpu.CompilerParams(
            dimension_semantics=("parallel", "arbitrary"),
        ),
    )(x, pos_embed_weight[:seq])
    return out
